# Initial kernel scaffold; baseline (speedup 1.0000x reference)
import jax
import jax.numpy as jnp
from jax.experimental import pallas as pl


def kernel(x, edge_index, W1, b1, W2, b2, W3, b3):
    raise NotImplementedError("write your pallas kernel here")


# speedup vs baseline: 26.8162x; 26.8162x over previous
import functools

import jax
import jax.numpy as jnp
from jax import lax
from jax.experimental import pallas as pl
from jax.experimental.pallas import tpu as pltpu
import jax.experimental.pallas.tpu_sc as plsc

N = 10000
D = 128
H = D // 2
NPAD = 10240
ROW = 128
BR = 1000


def _sc_mesh():
    return plsc.VectorSubcoreMesh(core_axis_name="c", subcore_axis_name="s")


def _deg_call(dstp, R):
    RW = R // 32
    GRPS = RW // 8
    seg = NPAD // 16

    def body(dst_hbm, deg0_hbm, deg1_hbm, deg_sh, didx_v, ones_v, zer_v):
        c = lax.axis_index("c")
        s = lax.axis_index("s")
        for i in range(ROW // 16):
            ones_v[pl.ds(i * 16, 16)] = jnp.ones((16,), jnp.float32)
        for i in range(seg // 16):
            zer_v[pl.ds(i * 16, 16)] = jnp.zeros((16,), jnp.float32)
        pltpu.sync_copy(zer_v, deg_sh.at[pl.ds(s * seg, seg)])
        plsc.subcore_barrier()
        w = c * 16 + s

        def grp(g, carry):
            row0 = w * RW + g * 8
            pltpu.sync_copy(dst_hbm.at[pl.ds(row0, 8)], didx_v)
            for j in range(8):
                pltpu.sync_copy(ones_v, deg_sh.at[didx_v.at[j]], add=True)
            return carry

        lax.fori_loop(0, GRPS, grp, 0)
        plsc.subcore_barrier()

        @pl.when(c == 0)
        def _():
            pltpu.sync_copy(deg_sh.at[pl.ds(s * seg, seg)],
                            deg0_hbm.at[pl.ds(s * seg, seg)])

        @pl.when(c == 1)
        def _():
            pltpu.sync_copy(deg_sh.at[pl.ds(s * seg, seg)],
                            deg1_hbm.at[pl.ds(s * seg, seg)])

    f = pl.kernel(
        body,
        out_type=(jax.ShapeDtypeStruct((NPAD,), jnp.float32),
                  jax.ShapeDtypeStruct((NPAD,), jnp.float32)),
        mesh=_sc_mesh(),
        compiler_params=pltpu.CompilerParams(use_tc_tiling_on_sc=False),
        scratch_types=(
            pltpu.VMEM_SHARED((NPAD,), jnp.float32),
            pltpu.VMEM((8, ROW), jnp.int32),
            pltpu.VMEM((ROW,), jnp.float32),
            pltpu.VMEM((seg,), jnp.float32),
        ),
    )
    return f(dstp)


def _edge_call(g, comb2, R2):
    CW = 80
    RW = R2 // 32
    K = RW // 8
    stg = 624
    tail = N - 16 * stg
    outr = NPAD // 16

    def body(g_hbm, comb_hbm, out0_hbm, out1_hbm,
             acc_sh, ibuf0, ibuf1, rows4,
             gsem0, gsem1, gsem2, gsem3, ssem0, ssem1, ssem2, ssem3,
             isem0, isem1):
        c = lax.axis_index("c")
        s = lax.axis_index("s")
        gsems = (gsem0, gsem1, gsem2, gsem3)
        ssems = (ssem0, ssem1, ssem2, ssem3)
        base = (c * 16 + s) * RW
        dummy_idx = comb_hbm.at[pl.ds(0, 4)]
        dummy_rows = g_hbm.at[pl.ds(0, CW)]

        off = s * stg
        pltpu.sync_copy(g_hbm.at[pl.ds(off, stg)],
                        acc_sh.at[pl.ds(off, stg)])

        @pl.when(s == 0)
        def _():
            pltpu.sync_copy(g_hbm.at[pl.ds(16 * stg, tail)],
                            acc_sh.at[pl.ds(16 * stg, tail)])

        plsc.subcore_barrier()

        pltpu.sync_copy(comb_hbm.at[pl.ds(base, 4)], ibuf0)
        pltpu.async_copy(comb_hbm.at[pl.ds(base + 4, 4)], ibuf1, isem1)
        pltpu.async_copy(g_hbm.at[ibuf0.at[0, 0]], rows4.at[0], gsem0)
        pltpu.async_copy(g_hbm.at[ibuf0.at[1, 0]], rows4.at[1], gsem1)

        def outer(k, carry):
            for j in range(8):
                b = j % 4
                nb = (j + 2) % 4
                ib = ibuf0 if j < 4 else ibuf1
                pltpu.make_async_copy(dummy_rows, rows4.at[b],
                                      gsems[b]).wait()
                if j == 2:
                    pltpu.make_async_copy(dummy_idx, ibuf1, isem1).wait()
                if j < 2:
                    @pl.when(k > 0)
                    def _():
                        pltpu.make_async_copy(dummy_rows, rows4.at[nb],
                                              ssems[nb]).wait()
                else:
                    pltpu.make_async_copy(dummy_rows, rows4.at[nb],
                                          ssems[nb]).wait()
                if j == 1:
                    @pl.when(k > 0)
                    def _():
                        pltpu.async_copy(
                            comb_hbm.at[pl.ds(base + 8 * k + 4, 4)],
                            ibuf1, isem1)
                if j == 5:
                    @pl.when(k < K - 1)
                    def _():
                        pltpu.async_copy(
                            comb_hbm.at[pl.ds(base + 8 * k + 8, 4)],
                            ibuf0, isem0)
                if j < 6:
                    gib = ibuf0 if j + 2 < 4 else ibuf1
                    pltpu.async_copy(g_hbm.at[gib.at[(j + 2) % 4, 0]],
                                     rows4.at[nb], gsems[nb])
                elif j == 6:
                    @pl.when(k < K - 1)
                    def _():
                        pltpu.make_async_copy(dummy_idx, ibuf0,
                                              isem0).wait()
                        pltpu.async_copy(g_hbm.at[ibuf0.at[0, 0]],
                                         rows4.at[nb], gsems[nb])
                else:
                    @pl.when(k < K - 1)
                    def _():
                        pltpu.async_copy(g_hbm.at[ibuf0.at[1, 0]],
                                         rows4.at[nb], gsems[nb])
                pltpu.async_copy(rows4.at[b], acc_sh.at[ib.at[j % 4, 1]],
                                 ssems[b], priority=1, add=True)
            return carry

        lax.fori_loop(0, K, outer, 0)
        pltpu.make_async_copy(dummy_rows, rows4.at[2], ssem2).wait()
        pltpu.make_async_copy(dummy_rows, rows4.at[3], ssem3).wait()
        plsc.subcore_barrier()

        @pl.when(c == 0)
        def _():
            pltpu.sync_copy(acc_sh.at[pl.ds(s * outr, outr)],
                            out0_hbm.at[pl.ds(s * outr, outr)])

        @pl.when(c == 1)
        def _():
            pltpu.sync_copy(acc_sh.at[pl.ds(s * outr, outr)],
                            out1_hbm.at[pl.ds(s * outr, outr)])

    f = pl.kernel(
        body,
        out_type=(jax.ShapeDtypeStruct((NPAD, D), jnp.float32),
                  jax.ShapeDtypeStruct((NPAD, D), jnp.float32)),
        mesh=_sc_mesh(),
        compiler_params=pltpu.CompilerParams(use_tc_tiling_on_sc=False),
        scratch_types=(
            pltpu.VMEM_SHARED((NPAD, D), jnp.float32),
            pltpu.VMEM((4, 2, CW), jnp.int32),
            pltpu.VMEM((4, 2, CW), jnp.int32),
            pltpu.VMEM((4, CW, D), jnp.float32),
            pltpu.SemaphoreType.DMA,
            pltpu.SemaphoreType.DMA,
            pltpu.SemaphoreType.DMA,
            pltpu.SemaphoreType.DMA,
            pltpu.SemaphoreType.DMA,
            pltpu.SemaphoreType.DMA,
            pltpu.SemaphoreType.DMA,
            pltpu.SemaphoreType.DMA,
            pltpu.SemaphoreType.DMA,
            pltpu.SemaphoreType.DMA,
        ),
    )
    return f(g, comb2)


def _tc_first(x, W, dis):
    def body(x_ref, w_ref, d_ref, g_ref):
        g = jnp.dot(x_ref[...], w_ref[...], preferred_element_type=jnp.float32)
        g_ref[...] = g * d_ref[...]

    return pl.pallas_call(
        body,
        grid=(N // BR,),
        in_specs=[
            pl.BlockSpec((BR, D), lambda i: (i, 0)),
            pl.BlockSpec((D, D), lambda i: (0, 0)),
            pl.BlockSpec((BR, 1), lambda i: (i, 0)),
        ],
        out_specs=pl.BlockSpec((BR, D), lambda i: (i, 0)),
        out_shape=jax.ShapeDtypeStruct((N, D), jnp.float32),
    )(x, W, dis)


def _tc_mid(a0, a1, g, dis, b, W):
    def body(a0_ref, a1_ref, g_ref, d_ref, b_ref, w_ref, o_ref):
        acc = a0_ref[...] + a1_ref[...] - g_ref[...]
        h = jnp.maximum(acc * d_ref[...] + b_ref[...], 0.0)
        gn = jnp.dot(h, w_ref[...], preferred_element_type=jnp.float32)
        o_ref[...] = gn * d_ref[...]

    return pl.pallas_call(
        body,
        grid=(N // BR,),
        in_specs=[
            pl.BlockSpec((BR, D), lambda i: (i, 0)),
            pl.BlockSpec((BR, D), lambda i: (i, 0)),
            pl.BlockSpec((BR, D), lambda i: (i, 0)),
            pl.BlockSpec((BR, 1), lambda i: (i, 0)),
            pl.BlockSpec((1, D), lambda i: (0, 0)),
            pl.BlockSpec((D, D), lambda i: (0, 0)),
        ],
        out_specs=pl.BlockSpec((BR, D), lambda i: (i, 0)),
        out_shape=jax.ShapeDtypeStruct((N, D), jnp.float32),
    )(a0, a1, g, dis, b, W)


def _tc_final(a0, a1, g, dis, b):
    def body(a0_ref, a1_ref, g_ref, d_ref, b_ref, o_ref):
        acc = a0_ref[...] + a1_ref[...] - g_ref[...]
        o_ref[...] = acc * d_ref[...] + b_ref[...]

    return pl.pallas_call(
        body,
        grid=(N // BR,),
        in_specs=[
            pl.BlockSpec((BR, D), lambda i: (i, 0)),
            pl.BlockSpec((BR, D), lambda i: (i, 0)),
            pl.BlockSpec((BR, D), lambda i: (i, 0)),
            pl.BlockSpec((BR, 1), lambda i: (i, 0)),
            pl.BlockSpec((1, D), lambda i: (0, 0)),
        ],
        out_specs=pl.BlockSpec((BR, D), lambda i: (i, 0)),
        out_shape=jax.ShapeDtypeStruct((N, D), jnp.float32),
    )(a0, a1, g, dis, b)


def kernel(x, edge_index, W1, b1, W2, b2, W3, b3):
    src = edge_index[0].astype(jnp.int32)
    dst = edge_index[1].astype(jnp.int32)
    E = src.shape[0]
    rows = (E + ROW - 1) // ROW
    R = ((rows + 255) // 256) * 256
    P = R * ROW - E
    ar = jnp.arange(P, dtype=jnp.int32)
    srcp = jnp.concatenate([src, ar % ROW]).reshape(R, ROW)
    dstp = jnp.concatenate([dst, N + ar % (NPAD - N)]).reshape(R, ROW)
    R2 = R * ROW // 80
    comb2 = jnp.stack([srcp.reshape(R2, 80), dstp.reshape(R2, 80)], axis=1)

    deg0, deg1 = _deg_call(dstp, R)
    deg = deg0[:N] + deg1[:N] + 1.0
    dis = lax.rsqrt(deg)[:, None]
    b1r, b2r, b3r = b1[None, :], b2[None, :], b3[None, :]

    g = _tc_first(x, W1, dis)
    a0, a1 = _edge_call(g, comb2, R2)
    g = _tc_mid(a0, a1, g, dis, b1r, W2)
    a0, a1 = _edge_call(g, comb2, R2)
    g = _tc_mid(a0, a1, g, dis, b2r, W3)
    a0, a1 = _edge_call(g, comb2, R2)
    return _tc_final(a0, a1, g, dis, b3r)
